# baseline (speedup 1.0000x reference)
import functools

import jax
import jax.numpy as jnp
from jax.experimental import pallas as pl
from jax.experimental.pallas import tpu as pltpu

_TB = 1024


def _router_body(hs_ref, w_ref, w1_ref, w2_ref, i1_ref, i2_ref):
    hs = hs_ref[...]
    w = w_ref[...]
    logits = jax.lax.dot_general(
        hs, w, (((1,), (1,)), ((), ())), preferred_element_type=jnp.float32
    )
    idx = jax.lax.broadcasted_iota(jnp.int32, logits.shape, 1)
    m1 = jnp.max(logits, axis=1, keepdims=True)
    i1 = jnp.min(jnp.where(logits == m1, idx, 16), axis=1, keepdims=True)
    masked = jnp.where(idx == i1, -jnp.inf, logits)
    m2 = jnp.max(masked, axis=1, keepdims=True)
    i2 = jnp.min(jnp.where(masked == m2, idx, 16), axis=1, keepdims=True)
    t = jnp.exp(m2 - m1)
    denom = 1.0 + t
    w1_ref[...] = 1.0 / denom
    w2_ref[...] = t / denom
    i1_ref[...] = i1
    i2_ref[...] = i2


@functools.partial(jax.jit, static_argnames=())
def kernel(hidden_states, W):
    hs = hidden_states.reshape(-1, hidden_states.shape[-1])
    T, d = hs.shape
    E = W.shape[0]
    grid = (T // _TB,)
    out_shapes = (
        jax.ShapeDtypeStruct((T, 1), jnp.float32),
        jax.ShapeDtypeStruct((T, 1), jnp.float32),
        jax.ShapeDtypeStruct((T, 1), jnp.int32),
        jax.ShapeDtypeStruct((T, 1), jnp.int32),
    )
    col_spec = pl.BlockSpec((_TB, 1), lambda i: (i, 0))
    w1, w2, i1, i2 = pl.pallas_call(
        _router_body,
        grid=grid,
        in_specs=[
            pl.BlockSpec((_TB, d), lambda i: (i, 0)),
            pl.BlockSpec((E, d), lambda i: (0, 0)),
        ],
        out_specs=(col_spec, col_spec, col_spec, col_spec),
        out_shape=out_shapes,
    )(hs, W)
    top_weights = jnp.concatenate([w1, w2], axis=1)
    top_experts = jnp.concatenate([i1, i2], axis=1)
    return (top_weights, top_experts)
